# baseline (speedup 1.0000x reference)
import functools

import jax
import jax.numpy as jnp
from jax import lax
from jax.experimental import pallas as pl
from jax.experimental.pallas import tpu as pltpu
from jax.experimental.pallas import tpu_sc as plsc

_N = 10000
_D = 128
_LAT = 64
_NC = 2
_NS = 16
_NW = _NC * _NS
_CHUNK = 64
_IB = 16
_NBUF = 4
_NPAD = 10240
_RPT = _NPAD // _NS
_BLK = 1024
_F32 = jnp.float32


@functools.lru_cache(maxsize=None)
def _make_agg(k_chunks, with_counts):
  mesh = plsc.VectorSubcoreMesh(core_axis_name="c", subcore_axis_name="s")
  assert k_chunks % _IB == 0
  out_type = [jax.ShapeDtypeStruct((_NC, _NPAD, _D), _F32)]
  scratch = [
      pltpu.VMEM((3, _IB, _CHUNK), jnp.int32),
      pltpu.VMEM((3, _IB, _CHUNK), jnp.int32),
      pltpu.VMEM((_NBUF, _CHUNK, _D), _F32),
      pltpu.VMEM_SHARED((_NPAD, _D), _F32),
  ] + [pltpu.SemaphoreType.DMA] * (_NBUF * (3 if with_counts else 2))
  if with_counts:
    out_type.append(jax.ShapeDtypeStruct((_NC, _NPAD), _F32))
    scratch += [
        pltpu.VMEM((_RPT,), _F32),
        pltpu.VMEM_SHARED((_NPAD,), _F32),
    ]

  def body(*refs):
    csems = ()
    if with_counts:
      (src_h, dst_h, tbl_h, sum_h, cnt_h,
       src_v, dst_v, rows_v, acc_sh, *rest) = refs
      gsems = rest[:_NBUF]
      ssems = rest[_NBUF:2 * _NBUF]
      csems = rest[2 * _NBUF:3 * _NBUF]
      ones_v, cnt_sh = rest[3 * _NBUF:]
    else:
      (src_h, dst_h, tbl_h, sum_h,
       src_v, dst_v, rows_v, acc_sh, *rest) = refs
      gsems = rest[:_NBUF]
      ssems = rest[_NBUF:2 * _NBUF]
      cnt_h = ones_v = cnt_sh = None
    c = lax.axis_index("c")
    s = lax.axis_index("s")
    wid = c * _NS + s
    base = s * _RPT

    zv = jnp.zeros((16,), _F32)

    nl = _D // 16

    def _zrows(i, _):
      rows_v[0, i // nl, pl.ds((i % nl) * 16, 16)] = zv
      return 0
    lax.fori_loop(0, _CHUNK * nl, _zrows, 0)
    for b in range(_RPT // _CHUNK):
      pltpu.sync_copy(rows_v.at[0], acc_sh.at[pl.ds(base + b * _CHUNK, _CHUNK)])
    if with_counts:
      def _zones(i, _):
        ones_v[pl.ds(i * 16, 16)] = zv
        return 0
      lax.fori_loop(0, _RPT // 16, _zones, 0)
      pltpu.sync_copy(ones_v, cnt_sh.at[pl.ds(base, _RPT)])
      ov = jnp.ones((16,), _F32)

      def _ones(i, _):
        ones_v[pl.ds(i * 16, 16)] = ov
        return 0
      lax.fori_loop(0, _CHUNK // 16, _ones, 0)

    plsc.subcore_barrier()

    nb = k_chunks // _IB
    assert k_chunks % _NBUF == 0 and _NBUF >= 4

    def _ldidx(b, slot):
      pltpu.sync_copy(src_h.at[wid, pl.ds(b * _IB, _IB)], src_v.at[slot])
      pltpu.sync_copy(dst_h.at[wid, pl.ds(b * _IB, _IB)], dst_v.at[slot])

    def _gather(j, q):
      pltpu.async_copy(tbl_h.at[src_v.at[(j // _IB) % 3, j % _IB]],
                       rows_v.at[q], gsems[q])

    def _wait_gather(q):
      pltpu.make_async_copy(tbl_h.at[pl.ds(0, _CHUNK)],
                            rows_v.at[q], gsems[q]).wait()

    def _scatter(j, q):
      idx = dst_v.at[(j // _IB) % 3, j % _IB]
      pltpu.async_copy(rows_v.at[q], acc_sh.at[idx], ssems[q], add=True)
      if with_counts:
        pltpu.async_copy(ones_v.at[pl.ds(0, _CHUNK)], cnt_sh.at[idx],
                         csems[q], add=True)

    def _wait_scatter(q):
      pltpu.make_async_copy(rows_v.at[q], acc_sh.at[pl.ds(0, _CHUNK)],
                            ssems[q]).wait()
      if with_counts:
        pltpu.make_async_copy(ones_v.at[pl.ds(0, _CHUNK)],
                              cnt_sh.at[pl.ds(0, _CHUNK)], csems[q]).wait()

    _ldidx(0, 0)
    for q in range(_NBUF):
      _gather(q, q)

    def group(g, _):
      j0 = g * _NBUF
      b0 = j0 // _IB

      @pl.when(jnp.logical_and(j0 % _IB == 0, b0 + 1 < nb))
      def _():
        _ldidx(b0 + 1, (b0 + 1) % 3)

      for p in range(_NBUF):
        j = j0 + p
        _wait_gather(p)
        _scatter(j, p)
        qq = (p + 2) % _NBUF

        @pl.when(jnp.logical_and(j >= 2, j + 2 < k_chunks))
        def _():
          _wait_scatter(qq)
          _gather(j + 2, qq)
      return 0
    lax.fori_loop(0, k_chunks // _NBUF, group, 0)

    for q in range(_NBUF):
      _wait_scatter(q)

    plsc.subcore_barrier()

    for b in range(_RPT // _CHUNK):
      pltpu.sync_copy(acc_sh.at[pl.ds(base + b * _CHUNK, _CHUNK)], rows_v.at[0])
      pltpu.sync_copy(rows_v.at[0], sum_h.at[c, pl.ds(base + b * _CHUNK, _CHUNK)])
    if with_counts:
      pltpu.sync_copy(cnt_sh.at[pl.ds(base, _RPT)], ones_v)
      pltpu.sync_copy(ones_v, cnt_h.at[c, pl.ds(base, _RPT)])

  return pl.kernel(body, out_type=tuple(out_type), mesh=mesh,
                   scratch_types=tuple(scratch))


def _mean_scaled(cr, s):
  cnt = cr[0:1, :] + cr[1:2, :]
  inv = 1.0 / jnp.maximum(cnt, 1.0)
  ii = lax.broadcasted_iota(jnp.int32, (_D, _D), 0)
  jj = lax.broadcasted_iota(jnp.int32, (_D, _D), 1)
  eye = ii == jj
  outs = []
  for k in range(_BLK // _D):
    dk = jnp.where(eye, jnp.broadcast_to(inv[:, k * _D:(k + 1) * _D], (_D, _D)), 0.0)
    outs.append(jnp.dot(dk, s[k * _D:(k + 1) * _D, :], preferred_element_type=_F32))
  return jnp.concatenate(outs, axis=0)


def _tc1_body(s1r, cr, xr,
              wl1a, wr1a, bl1a, g1a, b1a, wl2a, wr2a, bl2a,
              wl1b, wr1b, bl1b, g1b, b1b, wl2b, wr2b, bl2b,
              p_out, ra_out, rb_out):
  mean1 = _mean_scaled(cr[:], s1r[0] + s1r[1])
  xb = xr[:]
  for wl1, wr1, bl1, g1, b1, wl2, wr2, bl2, r_out, lo in (
      (wl1a, wr1a, bl1a, g1a, b1a, wl2a, wr2a, bl2a, ra_out, 0),
      (wl1b, wr1b, bl1b, g1b, b1b, wl2b, wr2b, bl2b, rb_out, _LAT)):
    h = (jnp.dot(mean1, wl1[:], preferred_element_type=_F32)
         + jnp.dot(xb, wr1[:], preferred_element_type=_F32) + bl1[:])
    m = jnp.mean(h, axis=1, keepdims=True)
    v = jnp.mean((h - m) * (h - m), axis=1, keepdims=True)
    hr = jnp.maximum((h - m) * lax.rsqrt(v + 1e-5) * g1[:] + b1[:], 0.0)
    p_out[:, lo:lo + _LAT] = jnp.dot(hr, wl2[:], preferred_element_type=_F32)
    r_out[:] = jnp.dot(hr, wr2[:], preferred_element_type=_F32) + bl2[:]


def _tc2_body(s2r, cr, ra, rb, mu_out, lv_out):
  mean2 = _mean_scaled(cr[:], s2r[0] + s2r[1])
  mu_out[:] = mean2[:, :_LAT] + ra[:]
  lv_out[:] = mean2[:, _LAT:] + rb[:]


def kernel(x, edge_index, Wl1_mu, bl1_mu, Wr1_mu, g1_mu, b1_mu, Wl2_mu,
           bl2_mu, Wr2_mu, Wl1_lv, bl1_lv, Wr1_lv, g1_lv, b1_lv, Wl2_lv,
           bl2_lv, Wr2_lv):
  src = edge_index[0].astype(jnp.int32)
  dst = edge_index[1].astype(jnp.int32)
  e = src.shape[0]
  k_chunks = -(-e // (_NW * _CHUNK))
  k_chunks = -(-k_chunks // _IB) * _IB
  pad = _NW * _CHUNK * k_chunks - e
  if pad:
    ar = lax.iota(jnp.int32, pad)
    src = jnp.concatenate([src, (ar * 7919) % _N])
    dst = jnp.concatenate([dst, _N + (ar % (_NPAD - _N))])
  srcw = src.reshape(_NW, k_chunks, _CHUNK)
  dstw = dst.reshape(_NW, k_chunks, _CHUNK)

  sum1, cnt = _make_agg(k_chunks, True)(srcw, dstw, x)

  grid = (_NPAD // _BLK,)
  row_d = pl.BlockSpec((_BLK, _D), lambda i: (i, 0))
  row_l = pl.BlockSpec((_BLK, _LAT), lambda i: (i, 0))
  part = pl.BlockSpec((2, _BLK, _D), lambda i: (0, i, 0))
  cnt_s = pl.BlockSpec((2, _BLK), lambda i: (0, i))
  w_dd = pl.BlockSpec((_D, _D), lambda i: (0, 0))
  w_dl = pl.BlockSpec((_D, _LAT), lambda i: (0, 0))
  v_d = pl.BlockSpec((1, _D), lambda i: (0, 0))
  v_l = pl.BlockSpec((1, _LAT), lambda i: (0, 0))

  tower_w = []
  for (wl1, bl1, wr1, g1, b1, wl2, bl2, wr2) in (
      (Wl1_mu, bl1_mu, Wr1_mu, g1_mu, b1_mu, Wl2_mu, bl2_mu, Wr2_mu),
      (Wl1_lv, bl1_lv, Wr1_lv, g1_lv, b1_lv, Wl2_lv, bl2_lv, Wr2_lv)):
    tower_w += [wl1.T, wr1.T, bl1.reshape(1, _D), g1.reshape(1, _D),
                b1.reshape(1, _D), wl2.T, wr2.T, bl2.reshape(1, _LAT)]
  tower_specs = [w_dd, w_dd, v_d, v_d, v_d, w_dl, w_dl, v_l] * 2

  p, r_mu, r_lv = pl.pallas_call(
      _tc1_body,
      grid=grid,
      in_specs=[part, cnt_s, row_d] + tower_specs,
      out_specs=[row_d, row_l, row_l],
      out_shape=[
          jax.ShapeDtypeStruct((_N, _D), _F32),
          jax.ShapeDtypeStruct((_N, _LAT), _F32),
          jax.ShapeDtypeStruct((_N, _LAT), _F32),
      ],
  )(sum1, cnt, x, *tower_w)

  (sum2,) = _make_agg(k_chunks, False)(srcw, dstw, p)

  mu, lv = pl.pallas_call(
      _tc2_body,
      grid=grid,
      in_specs=[part, cnt_s, row_l, row_l],
      out_specs=[row_l, row_l],
      out_shape=[
          jax.ShapeDtypeStruct((_N, _LAT), _F32),
          jax.ShapeDtypeStruct((_N, _LAT), _F32),
      ],
  )(sum2, cnt, r_mu, r_lv)

  return (mu, lv)


# speedup vs baseline: 1.1178x; 1.1178x over previous
import functools

import jax
import jax.numpy as jnp
from jax import lax
from jax.experimental import pallas as pl
from jax.experimental.pallas import tpu as pltpu
from jax.experimental.pallas import tpu_sc as plsc

_N = 10000
_D = 128
_LAT = 64
_NC = 2
_NS = 16
_NW = _NC * _NS
_CHUNK = 64
_IB = 16
_NBUF = 4
_NPAD = 10240
_RPT = _NPAD // _NS
_BLK = 1024
_F32 = jnp.float32


@functools.lru_cache(maxsize=None)
def _make_agg(k_chunks, with_counts):
  mesh = plsc.VectorSubcoreMesh(core_axis_name="c", subcore_axis_name="s")
  assert k_chunks % _IB == 0
  out_type = [jax.ShapeDtypeStruct((_NC, _NPAD, _D), _F32)]
  scratch = [
      pltpu.VMEM((3, _IB, _CHUNK), jnp.int32),
      pltpu.VMEM((3, _IB, _CHUNK), jnp.int32),
      pltpu.VMEM((_NBUF, _CHUNK, _D), _F32),
      pltpu.VMEM_SHARED((_NPAD, _D), _F32),
  ] + [pltpu.SemaphoreType.DMA] * (_NBUF * (3 if with_counts else 2))
  if with_counts:
    out_type.append(jax.ShapeDtypeStruct((_NC, _NPAD), _F32))
    scratch += [
        pltpu.VMEM((_RPT,), _F32),
        pltpu.VMEM_SHARED((_NPAD,), _F32),
    ]

  def body(*refs):
    csems = ()
    if with_counts:
      (src_h, dst_h, tbl_h, sum_h, cnt_h,
       src_v, dst_v, rows_v, acc_sh, *rest) = refs
      gsems = rest[:_NBUF]
      ssems = rest[_NBUF:2 * _NBUF]
      csems = rest[2 * _NBUF:3 * _NBUF]
      ones_v, cnt_sh = rest[3 * _NBUF:]
    else:
      (src_h, dst_h, tbl_h, sum_h,
       src_v, dst_v, rows_v, acc_sh, *rest) = refs
      gsems = rest[:_NBUF]
      ssems = rest[_NBUF:2 * _NBUF]
      cnt_h = ones_v = cnt_sh = None
    c = lax.axis_index("c")
    s = lax.axis_index("s")
    wid = c * _NS + s
    base = s * _RPT

    zv = jnp.zeros((16,), _F32)

    nl = _D // 16

    def _zrows(i, _):
      rows_v[0, i // nl, pl.ds((i % nl) * 16, 16)] = zv
      return 0
    lax.fori_loop(0, _CHUNK * nl, _zrows, 0)
    for b in range(_RPT // _CHUNK):
      pltpu.sync_copy(rows_v.at[0], acc_sh.at[pl.ds(base + b * _CHUNK, _CHUNK)])
    if with_counts:
      def _zones(i, _):
        ones_v[pl.ds(i * 16, 16)] = zv
        return 0
      lax.fori_loop(0, _RPT // 16, _zones, 0)
      pltpu.sync_copy(ones_v, cnt_sh.at[pl.ds(base, _RPT)])
      ov = jnp.ones((16,), _F32)

      def _ones(i, _):
        ones_v[pl.ds(i * 16, 16)] = ov
        return 0
      lax.fori_loop(0, _CHUNK // 16, _ones, 0)

    plsc.subcore_barrier()

    nb = k_chunks // _IB
    assert k_chunks % _NBUF == 0 and _NBUF >= 4

    def _ldidx(b, slot):
      pltpu.sync_copy(src_h.at[wid, pl.ds(b * _IB, _IB)], src_v.at[slot])
      pltpu.sync_copy(dst_h.at[wid, pl.ds(b * _IB, _IB)], dst_v.at[slot])

    def _gather(j, q):
      pltpu.async_copy(tbl_h.at[src_v.at[(j // _IB) % 3, j % _IB]],
                       rows_v.at[q], gsems[q])

    def _wait_gather(q):
      pltpu.make_async_copy(tbl_h.at[pl.ds(0, _CHUNK)],
                            rows_v.at[q], gsems[q]).wait()

    def _scatter(j, q):
      idx = dst_v.at[(j // _IB) % 3, j % _IB]
      pltpu.async_copy(rows_v.at[q], acc_sh.at[idx], ssems[q], add=True)
      if with_counts:
        pltpu.async_copy(ones_v.at[pl.ds(0, _CHUNK)], cnt_sh.at[idx],
                         csems[q], add=True)

    def _wait_scatter(q):
      pltpu.make_async_copy(rows_v.at[q], acc_sh.at[pl.ds(0, _CHUNK)],
                            ssems[q]).wait()
      if with_counts:
        pltpu.make_async_copy(ones_v.at[pl.ds(0, _CHUNK)],
                              cnt_sh.at[pl.ds(0, _CHUNK)], csems[q]).wait()

    _ldidx(0, 0)
    for q in range(_NBUF):
      _gather(q, q)

    def group(g, _):
      j0 = g * _NBUF
      b0 = j0 // _IB

      @pl.when(jnp.logical_and(j0 % _IB == 0, b0 + 1 < nb))
      def _():
        _ldidx(b0 + 1, (b0 + 1) % 3)

      for p in range(_NBUF):
        j = j0 + p
        _wait_gather(p)
        _scatter(j, p)
        qq = (p + 3) % _NBUF

        @pl.when(jnp.logical_and(j >= 1, j + 3 < k_chunks))
        def _():
          _wait_scatter(qq)
          _gather(j + 3, qq)
      return 0
    lax.fori_loop(0, k_chunks // _NBUF, group, 0)

    for q in range(_NBUF):
      _wait_scatter(q)

    plsc.subcore_barrier()

    for b in range(_RPT // _CHUNK):
      pltpu.sync_copy(acc_sh.at[pl.ds(base + b * _CHUNK, _CHUNK)], rows_v.at[0])
      pltpu.sync_copy(rows_v.at[0], sum_h.at[c, pl.ds(base + b * _CHUNK, _CHUNK)])
    if with_counts:
      pltpu.sync_copy(cnt_sh.at[pl.ds(base, _RPT)], ones_v)
      pltpu.sync_copy(ones_v, cnt_h.at[c, pl.ds(base, _RPT)])

  return pl.kernel(body, out_type=tuple(out_type), mesh=mesh,
                   scratch_types=tuple(scratch))


def _mean_scaled(cr, s):
  cnt = cr[0:1, :] + cr[1:2, :]
  inv = 1.0 / jnp.maximum(cnt, 1.0)
  ii = lax.broadcasted_iota(jnp.int32, (_D, _D), 0)
  jj = lax.broadcasted_iota(jnp.int32, (_D, _D), 1)
  eye = ii == jj
  outs = []
  for k in range(_BLK // _D):
    dk = jnp.where(eye, jnp.broadcast_to(inv[:, k * _D:(k + 1) * _D], (_D, _D)), 0.0)
    outs.append(jnp.dot(dk, s[k * _D:(k + 1) * _D, :], preferred_element_type=_F32))
  return jnp.concatenate(outs, axis=0)


def _tc1_body(s1r, cr, xr,
              wl1a, wr1a, bl1a, g1a, b1a, wl2a, wr2a, bl2a,
              wl1b, wr1b, bl1b, g1b, b1b, wl2b, wr2b, bl2b,
              p_out, ra_out, rb_out):
  mean1 = _mean_scaled(cr[:], s1r[0] + s1r[1])
  xb = xr[:]
  for wl1, wr1, bl1, g1, b1, wl2, wr2, bl2, r_out, lo in (
      (wl1a, wr1a, bl1a, g1a, b1a, wl2a, wr2a, bl2a, ra_out, 0),
      (wl1b, wr1b, bl1b, g1b, b1b, wl2b, wr2b, bl2b, rb_out, _LAT)):
    h = (jnp.dot(mean1, wl1[:], preferred_element_type=_F32)
         + jnp.dot(xb, wr1[:], preferred_element_type=_F32) + bl1[:])
    m = jnp.mean(h, axis=1, keepdims=True)
    v = jnp.mean((h - m) * (h - m), axis=1, keepdims=True)
    hr = jnp.maximum((h - m) * lax.rsqrt(v + 1e-5) * g1[:] + b1[:], 0.0)
    p_out[:, lo:lo + _LAT] = jnp.dot(hr, wl2[:], preferred_element_type=_F32)
    r_out[:] = jnp.dot(hr, wr2[:], preferred_element_type=_F32) + bl2[:]


def _tc2_body(s2r, cr, ra, rb, mu_out, lv_out):
  mean2 = _mean_scaled(cr[:], s2r[0] + s2r[1])
  mu_out[:] = mean2[:, :_LAT] + ra[:]
  lv_out[:] = mean2[:, _LAT:] + rb[:]


def kernel(x, edge_index, Wl1_mu, bl1_mu, Wr1_mu, g1_mu, b1_mu, Wl2_mu,
           bl2_mu, Wr2_mu, Wl1_lv, bl1_lv, Wr1_lv, g1_lv, b1_lv, Wl2_lv,
           bl2_lv, Wr2_lv):
  src = edge_index[0].astype(jnp.int32)
  dst = edge_index[1].astype(jnp.int32)
  e = src.shape[0]
  k_chunks = -(-e // (_NW * _CHUNK))
  k_chunks = -(-k_chunks // _IB) * _IB
  pad = _NW * _CHUNK * k_chunks - e
  if pad:
    ar = lax.iota(jnp.int32, pad)
    src = jnp.concatenate([src, (ar * 7919) % _N])
    dst = jnp.concatenate([dst, _N + (ar % (_NPAD - _N))])
  srcw = src.reshape(_NW, k_chunks, _CHUNK)
  dstw = dst.reshape(_NW, k_chunks, _CHUNK)

  sum1, cnt = _make_agg(k_chunks, True)(srcw, dstw, x)

  grid = (_NPAD // _BLK,)
  row_d = pl.BlockSpec((_BLK, _D), lambda i: (i, 0))
  row_l = pl.BlockSpec((_BLK, _LAT), lambda i: (i, 0))
  part = pl.BlockSpec((2, _BLK, _D), lambda i: (0, i, 0))
  cnt_s = pl.BlockSpec((2, _BLK), lambda i: (0, i))
  w_dd = pl.BlockSpec((_D, _D), lambda i: (0, 0))
  w_dl = pl.BlockSpec((_D, _LAT), lambda i: (0, 0))
  v_d = pl.BlockSpec((1, _D), lambda i: (0, 0))
  v_l = pl.BlockSpec((1, _LAT), lambda i: (0, 0))

  tower_w = []
  for (wl1, bl1, wr1, g1, b1, wl2, bl2, wr2) in (
      (Wl1_mu, bl1_mu, Wr1_mu, g1_mu, b1_mu, Wl2_mu, bl2_mu, Wr2_mu),
      (Wl1_lv, bl1_lv, Wr1_lv, g1_lv, b1_lv, Wl2_lv, bl2_lv, Wr2_lv)):
    tower_w += [wl1.T, wr1.T, bl1.reshape(1, _D), g1.reshape(1, _D),
                b1.reshape(1, _D), wl2.T, wr2.T, bl2.reshape(1, _LAT)]
  tower_specs = [w_dd, w_dd, v_d, v_d, v_d, w_dl, w_dl, v_l] * 2

  p, r_mu, r_lv = pl.pallas_call(
      _tc1_body,
      grid=grid,
      in_specs=[part, cnt_s, row_d] + tower_specs,
      out_specs=[row_d, row_l, row_l],
      out_shape=[
          jax.ShapeDtypeStruct((_N, _D), _F32),
          jax.ShapeDtypeStruct((_N, _LAT), _F32),
          jax.ShapeDtypeStruct((_N, _LAT), _F32),
      ],
  )(sum1, cnt, x, *tower_w)

  (sum2,) = _make_agg(k_chunks, False)(srcw, dstw, p)

  mu, lv = pl.pallas_call(
      _tc2_body,
      grid=grid,
      in_specs=[part, cnt_s, row_l, row_l],
      out_specs=[row_l, row_l],
      out_shape=[
          jax.ShapeDtypeStruct((_N, _LAT), _F32),
          jax.ShapeDtypeStruct((_N, _LAT), _F32),
      ],
  )(sum2, cnt, r_mu, r_lv)

  return (mu, lv)


# speedup vs baseline: 1.1440x; 1.0235x over previous
import functools

import jax
import jax.numpy as jnp
from jax import lax
from jax.experimental import pallas as pl
from jax.experimental.pallas import tpu as pltpu
from jax.experimental.pallas import tpu_sc as plsc

_N = 10000
_D = 128
_LAT = 64
_NC = 2
_NS = 16
_NW = _NC * _NS
_CHUNK = 64
_IB = 16
_NBUF = 4
_NPAD = 10240
_RPT = _NPAD // _NS
_BLK = 1024
_F32 = jnp.float32


@functools.lru_cache(maxsize=None)
def _make_agg(k_chunks, with_counts):
  mesh = plsc.VectorSubcoreMesh(core_axis_name="c", subcore_axis_name="s")
  assert k_chunks % _IB == 0
  out_type = [jax.ShapeDtypeStruct((_NC, _NPAD, _D), _F32)]
  scratch = [
      pltpu.VMEM((3, _IB, _CHUNK), jnp.int32),
      pltpu.VMEM((3, _IB, _CHUNK), jnp.int32),
      pltpu.VMEM((_NBUF, _CHUNK, _D), _F32),
      pltpu.VMEM_SHARED((_NPAD, _D), _F32),
  ] + [pltpu.SemaphoreType.DMA] * _NBUF
  if with_counts:
    out_type.append(jax.ShapeDtypeStruct((_NC, _NPAD), _F32))
    scratch += [
        pltpu.VMEM((_RPT,), _F32),
        pltpu.VMEM_SHARED((_NPAD,), _F32),
    ]

  def body(*refs):
    if with_counts:
      (src_h, dst_h, tbl_h, sum_h, cnt_h,
       src_v, dst_v, rows_v, acc_sh, *rest) = refs
      gsems = rest[:_NBUF]
      ones_v, cnt_sh = rest[_NBUF:]
    else:
      (src_h, dst_h, tbl_h, sum_h,
       src_v, dst_v, rows_v, acc_sh, *gsems) = refs
      cnt_h = ones_v = cnt_sh = None
    c = lax.axis_index("c")
    s = lax.axis_index("s")
    wid = c * _NS + s
    base = s * _RPT

    zv = jnp.zeros((16,), _F32)

    nl = _D // 16

    def _zrows(i, _):
      rows_v[0, i // nl, pl.ds((i % nl) * 16, 16)] = zv
      return 0
    lax.fori_loop(0, _CHUNK * nl, _zrows, 0)
    for b in range(_RPT // _CHUNK):
      pltpu.sync_copy(rows_v.at[0], acc_sh.at[pl.ds(base + b * _CHUNK, _CHUNK)])
    if with_counts:
      def _zones(i, _):
        ones_v[pl.ds(i * 16, 16)] = zv
        return 0
      lax.fori_loop(0, _RPT // 16, _zones, 0)
      pltpu.sync_copy(ones_v, cnt_sh.at[pl.ds(base, _RPT)])
      ov = jnp.ones((16,), _F32)

      def _ones(i, _):
        ones_v[pl.ds(i * 16, 16)] = ov
        return 0
      lax.fori_loop(0, _CHUNK // 16, _ones, 0)

    plsc.subcore_barrier()

    nb = k_chunks // _IB
    assert k_chunks % _NBUF == 0 and _NBUF >= 4

    def _ldidx(b, slot):
      pltpu.sync_copy(src_h.at[wid, pl.ds(b * _IB, _IB)], src_v.at[slot])
      pltpu.sync_copy(dst_h.at[wid, pl.ds(b * _IB, _IB)], dst_v.at[slot])

    def _gather(j, q):
      pltpu.async_copy(tbl_h.at[src_v.at[(j // _IB) % 3, j % _IB]],
                       rows_v.at[q], gsems[q])

    def _wait_gather(q):
      pltpu.make_async_copy(tbl_h.at[pl.ds(0, _CHUNK)],
                            rows_v.at[q], gsems[q]).wait()

    def _scatter(j, q):
      idx = dst_v.at[(j // _IB) % 3, j % _IB]
      pltpu.sync_copy(rows_v.at[q], acc_sh.at[idx], add=True)
      if with_counts:
        pltpu.sync_copy(ones_v.at[pl.ds(0, _CHUNK)], cnt_sh.at[idx], add=True)

    _ldidx(0, 0)
    for q in range(_NBUF):
      _gather(q, q)

    def group(g, _):
      j0 = g * _NBUF
      b0 = j0 // _IB

      @pl.when(jnp.logical_and(j0 % _IB == 0, b0 + 1 < nb))
      def _():
        _ldidx(b0 + 1, (b0 + 1) % 3)

      for p in range(_NBUF):
        j = j0 + p
        _wait_gather(p)
        _scatter(j, p)

        @pl.when(j + _NBUF < k_chunks)
        def _():
          _gather(j + _NBUF, p)
      return 0
    lax.fori_loop(0, k_chunks // _NBUF, group, 0)

    plsc.subcore_barrier()

    for b in range(_RPT // _CHUNK):
      pltpu.sync_copy(acc_sh.at[pl.ds(base + b * _CHUNK, _CHUNK)], rows_v.at[0])
      pltpu.sync_copy(rows_v.at[0], sum_h.at[c, pl.ds(base + b * _CHUNK, _CHUNK)])
    if with_counts:
      pltpu.sync_copy(cnt_sh.at[pl.ds(base, _RPT)], ones_v)
      pltpu.sync_copy(ones_v, cnt_h.at[c, pl.ds(base, _RPT)])

  return pl.kernel(body, out_type=tuple(out_type), mesh=mesh,
                   scratch_types=tuple(scratch))


def _mean_scaled(cr, s):
  cnt = cr[0:1, :] + cr[1:2, :]
  inv = 1.0 / jnp.maximum(cnt, 1.0)
  ii = lax.broadcasted_iota(jnp.int32, (_D, _D), 0)
  jj = lax.broadcasted_iota(jnp.int32, (_D, _D), 1)
  eye = ii == jj
  outs = []
  for k in range(_BLK // _D):
    dk = jnp.where(eye, jnp.broadcast_to(inv[:, k * _D:(k + 1) * _D], (_D, _D)), 0.0)
    outs.append(jnp.dot(dk, s[k * _D:(k + 1) * _D, :], preferred_element_type=_F32))
  return jnp.concatenate(outs, axis=0)


def _dot_t(a, w):
  return lax.dot_general(a, w, (((1,), (1,)), ((), ())),
                         preferred_element_type=_F32)


def _tc1_body(s1r, cr, xr,
              wl1a, wr1a, bl1a, g1a, b1a, wl2a, wr2a, bl2a,
              wl1b, wr1b, bl1b, g1b, b1b, wl2b, wr2b, bl2b,
              p_out, ra_out, rb_out):
  mean1 = _mean_scaled(cr[:], s1r[0] + s1r[1])
  xb = xr[:]
  for wl1, wr1, bl1, g1, b1, wl2, wr2, bl2, r_out, lo in (
      (wl1a, wr1a, bl1a, g1a, b1a, wl2a, wr2a, bl2a, ra_out, 0),
      (wl1b, wr1b, bl1b, g1b, b1b, wl2b, wr2b, bl2b, rb_out, _LAT)):
    h = _dot_t(mean1, wl1[:]) + _dot_t(xb, wr1[:]) + bl1[:]
    m = jnp.mean(h, axis=1, keepdims=True)
    v = jnp.mean((h - m) * (h - m), axis=1, keepdims=True)
    hr = jnp.maximum((h - m) * lax.rsqrt(v + 1e-5) * g1[:] + b1[:], 0.0)
    p_out[:, lo:lo + _LAT] = _dot_t(hr, wl2[:])
    r_out[:] = _dot_t(hr, wr2[:]) + bl2[:]


def _tc2_body(s2r, cr, ra, rb, mu_out, lv_out):
  mean2 = _mean_scaled(cr[:], s2r[0] + s2r[1])
  mu_out[:] = mean2[:, :_LAT] + ra[:]
  lv_out[:] = mean2[:, _LAT:] + rb[:]


def kernel(x, edge_index, Wl1_mu, bl1_mu, Wr1_mu, g1_mu, b1_mu, Wl2_mu,
           bl2_mu, Wr2_mu, Wl1_lv, bl1_lv, Wr1_lv, g1_lv, b1_lv, Wl2_lv,
           bl2_lv, Wr2_lv):
  src = edge_index[0].astype(jnp.int32)
  dst = edge_index[1].astype(jnp.int32)
  e = src.shape[0]
  k_chunks = -(-e // (_NW * _CHUNK))
  k_chunks = -(-k_chunks // _IB) * _IB
  pad = _NW * _CHUNK * k_chunks - e
  if pad:
    ar = lax.iota(jnp.int32, pad)
    src = jnp.concatenate([src, (ar * 7919) % _N])
    dst = jnp.concatenate([dst, _N + (ar % (_NPAD - _N))])
  srcw = src.reshape(_NW, k_chunks, _CHUNK)
  dstw = dst.reshape(_NW, k_chunks, _CHUNK)

  sum1, cnt = _make_agg(k_chunks, True)(srcw, dstw, x)

  grid = (_NPAD // _BLK,)
  row_d = pl.BlockSpec((_BLK, _D), lambda i: (i, 0))
  row_l = pl.BlockSpec((_BLK, _LAT), lambda i: (i, 0))
  part = pl.BlockSpec((2, _BLK, _D), lambda i: (0, i, 0))
  cnt_s = pl.BlockSpec((2, _BLK), lambda i: (0, i))
  w_dd = pl.BlockSpec((_D, _D), lambda i: (0, 0))
  w_dl = pl.BlockSpec((_LAT, _D), lambda i: (0, 0))
  v_d = pl.BlockSpec((1, _D), lambda i: (0, 0))
  v_l = pl.BlockSpec((1, _LAT), lambda i: (0, 0))

  tower_w = []
  for (wl1, bl1, wr1, g1, b1, wl2, bl2, wr2) in (
      (Wl1_mu, bl1_mu, Wr1_mu, g1_mu, b1_mu, Wl2_mu, bl2_mu, Wr2_mu),
      (Wl1_lv, bl1_lv, Wr1_lv, g1_lv, b1_lv, Wl2_lv, bl2_lv, Wr2_lv)):
    tower_w += [wl1, wr1, bl1.reshape(1, _D), g1.reshape(1, _D),
                b1.reshape(1, _D), wl2, wr2, bl2.reshape(1, _LAT)]
  tower_specs = [w_dd, w_dd, v_d, v_d, v_d, w_dl, w_dl, v_l] * 2

  p, r_mu, r_lv = pl.pallas_call(
      _tc1_body,
      grid=grid,
      in_specs=[part, cnt_s, row_d] + tower_specs,
      out_specs=[row_d, row_l, row_l],
      out_shape=[
          jax.ShapeDtypeStruct((_N, _D), _F32),
          jax.ShapeDtypeStruct((_N, _LAT), _F32),
          jax.ShapeDtypeStruct((_N, _LAT), _F32),
      ],
  )(sum1, cnt, x, *tower_w)

  (sum2,) = _make_agg(k_chunks, False)(srcw, dstw, p)

  mu, lv = pl.pallas_call(
      _tc2_body,
      grid=grid,
      in_specs=[part, cnt_s, row_l, row_l],
      out_specs=[row_l, row_l],
      out_shape=[
          jax.ShapeDtypeStruct((_N, _LAT), _F32),
          jax.ShapeDtypeStruct((_N, _LAT), _F32),
      ],
  )(sum2, cnt, r_mu, r_lv)

  return (mu, lv)


# speedup vs baseline: 1.1565x; 1.0110x over previous
import functools

import jax
import jax.numpy as jnp
from jax import lax
from jax.experimental import pallas as pl
from jax.experimental.pallas import tpu as pltpu
from jax.experimental.pallas import tpu_sc as plsc

_N = 10000
_D = 128
_LAT = 64
_NC = 2
_NS = 16
_NW = _NC * _NS
_CHUNK = 64
_IB = 16
_NBUF = 4
_NPAD = 10240
_RPT = _NPAD // _NS
_BLK = 1024
_F32 = jnp.float32


@functools.lru_cache(maxsize=None)
def _make_agg(k_chunks, with_counts):
  mesh = plsc.VectorSubcoreMesh(core_axis_name="c", subcore_axis_name="s")
  assert k_chunks % _IB == 0
  out_type = [jax.ShapeDtypeStruct((_NC, _NPAD, _D), _F32)]
  scratch = [
      pltpu.VMEM((3, _IB, _CHUNK), jnp.int32),
      pltpu.VMEM((3, _IB, _CHUNK), jnp.int32),
      pltpu.VMEM((_NBUF, _CHUNK, _D), _F32),
      pltpu.VMEM_SHARED((_NPAD, _D), _F32),
  ] + [pltpu.SemaphoreType.DMA] * _NBUF
  if with_counts:
    out_type.append(jax.ShapeDtypeStruct((_NC, _NPAD), _F32))
    scratch += [
        pltpu.VMEM((_RPT,), _F32),
        pltpu.VMEM_SHARED((_NPAD,), _F32),
    ]

  def body(*refs):
    if with_counts:
      (src_h, dst_h, tbl_h, sum_h, cnt_h,
       src_v, dst_v, rows_v, acc_sh, *rest) = refs
      gsems = rest[:_NBUF]
      ones_v, cnt_sh = rest[_NBUF:]
    else:
      (src_h, dst_h, tbl_h, sum_h,
       src_v, dst_v, rows_v, acc_sh, *gsems) = refs
      cnt_h = ones_v = cnt_sh = None
    c = lax.axis_index("c")
    s = lax.axis_index("s")
    wid = c * _NS + s
    base = s * _RPT

    zv = jnp.zeros((16,), _F32)

    nl = _D // 16

    def _zrows(i, _):
      rows_v[0, i // nl, pl.ds((i % nl) * 16, 16)] = zv
      return 0
    lax.fori_loop(0, _CHUNK * nl, _zrows, 0)
    for b in range(_RPT // _CHUNK):
      pltpu.sync_copy(rows_v.at[0], acc_sh.at[pl.ds(base + b * _CHUNK, _CHUNK)])
    if with_counts:
      def _zones(i, _):
        ones_v[pl.ds(i * 16, 16)] = zv
        return 0
      lax.fori_loop(0, _RPT // 16, _zones, 0)
      pltpu.sync_copy(ones_v, cnt_sh.at[pl.ds(base, _RPT)])
      ov = jnp.ones((16,), _F32)

      def _ones(i, _):
        ones_v[pl.ds(i * 16, 16)] = ov
        return 0
      lax.fori_loop(0, _CHUNK // 16, _ones, 0)

    plsc.subcore_barrier()

    nb = k_chunks // _IB
    assert k_chunks % _NBUF == 0 and _NBUF >= 4

    def _ldidx(b, slot):
      pltpu.sync_copy(src_h.at[wid, pl.ds(b * _IB, _IB)], src_v.at[slot])
      pltpu.sync_copy(dst_h.at[wid, pl.ds(b * _IB, _IB)], dst_v.at[slot])

    def _gather(j, q):
      pltpu.async_copy(tbl_h.at[src_v.at[(j // _IB) % 3, j % _IB]],
                       rows_v.at[q], gsems[q])

    def _wait_gather(q):
      pltpu.make_async_copy(tbl_h.at[pl.ds(0, _CHUNK)],
                            rows_v.at[q], gsems[q]).wait()

    def _scatter(j, q):
      idx = dst_v.at[(j // _IB) % 3, j % _IB]
      pltpu.sync_copy(rows_v.at[q], acc_sh.at[idx], add=True)
      if with_counts:
        pltpu.sync_copy(ones_v.at[pl.ds(0, _CHUNK)], cnt_sh.at[idx], add=True)

    _ldidx(0, 0)
    for q in range(_NBUF):
      _gather(q, q)

    def group(g, _):
      j0 = g * _NBUF
      b0 = j0 // _IB

      @pl.when(jnp.logical_and(j0 % _IB == 0, b0 + 1 < nb))
      def _():
        _ldidx(b0 + 1, (b0 + 1) % 3)

      for p in range(_NBUF):
        j = j0 + p
        _wait_gather(p)
        _scatter(j, p)

        @pl.when(j + _NBUF < k_chunks)
        def _():
          _gather(j + _NBUF, p)
      return 0
    lax.fori_loop(0, k_chunks // _NBUF, group, 0)

    plsc.subcore_barrier()

    for b in range(_RPT // _CHUNK):
      pltpu.sync_copy(acc_sh.at[pl.ds(base + b * _CHUNK, _CHUNK)], rows_v.at[0])
      pltpu.sync_copy(rows_v.at[0], sum_h.at[c, pl.ds(base + b * _CHUNK, _CHUNK)])
    if with_counts:
      pltpu.sync_copy(cnt_sh.at[pl.ds(base, _RPT)], ones_v)
      pltpu.sync_copy(ones_v, cnt_h.at[c, pl.ds(base, _RPT)])

  return pl.kernel(body, out_type=tuple(out_type), mesh=mesh,
                   scratch_types=tuple(scratch))


def _mean_scaled(cr, s):
  cnt = cr[0:1, :] + cr[1:2, :]
  inv = 1.0 / jnp.maximum(cnt, 1.0)
  ii = lax.broadcasted_iota(jnp.int32, (_D, _D), 0)
  jj = lax.broadcasted_iota(jnp.int32, (_D, _D), 1)
  eye = ii == jj
  outs = []
  for k in range(_BLK // _D):
    dk = jnp.where(eye, jnp.broadcast_to(inv[:, k * _D:(k + 1) * _D], (_D, _D)), 0.0)
    outs.append(jnp.dot(dk, s[k * _D:(k + 1) * _D, :], preferred_element_type=_F32))
  return jnp.concatenate(outs, axis=0)


def _tc1_body(s1r, cr, xr,
              wl1a, wr1a, bl1a, g1a, b1a, wl2a, wr2a, bl2a,
              wl1b, wr1b, bl1b, g1b, b1b, wl2b, wr2b, bl2b,
              p_out, ra_out, rb_out):
  mean1 = _mean_scaled(cr[:], s1r[0] + s1r[1])
  xb = xr[:]
  for wl1, wr1, bl1, g1, b1, wl2, wr2, bl2, r_out, lo in (
      (wl1a, wr1a, bl1a, g1a, b1a, wl2a, wr2a, bl2a, ra_out, 0),
      (wl1b, wr1b, bl1b, g1b, b1b, wl2b, wr2b, bl2b, rb_out, _LAT)):
    h = (jnp.dot(mean1, wl1[:], preferred_element_type=_F32)
         + jnp.dot(xb, wr1[:], preferred_element_type=_F32) + bl1[:])
    m = jnp.mean(h, axis=1, keepdims=True)
    v = jnp.mean((h - m) * (h - m), axis=1, keepdims=True)
    hr = jnp.maximum((h - m) * lax.rsqrt(v + 1e-5) * g1[:] + b1[:], 0.0)
    p_out[:, lo:lo + _LAT] = jnp.dot(hr, wl2[:], preferred_element_type=_F32)
    r_out[:] = jnp.dot(hr, wr2[:], preferred_element_type=_F32) + bl2[:]


def _tc2_body(s2r, cr, ra, rb, mu_out, lv_out):
  mean2 = _mean_scaled(cr[:], s2r[0] + s2r[1])
  mu_out[:] = mean2[:, :_LAT] + ra[:]
  lv_out[:] = mean2[:, _LAT:] + rb[:]


def kernel(x, edge_index, Wl1_mu, bl1_mu, Wr1_mu, g1_mu, b1_mu, Wl2_mu,
           bl2_mu, Wr2_mu, Wl1_lv, bl1_lv, Wr1_lv, g1_lv, b1_lv, Wl2_lv,
           bl2_lv, Wr2_lv):
  src = edge_index[0].astype(jnp.int32)
  dst = edge_index[1].astype(jnp.int32)
  e = src.shape[0]
  k_chunks = -(-e // (_NW * _CHUNK))
  k_chunks = -(-k_chunks // _IB) * _IB
  pad = _NW * _CHUNK * k_chunks - e
  if pad:
    ar = lax.iota(jnp.int32, pad)
    src = jnp.concatenate([src, (ar * 7919) % _N])
    dst = jnp.concatenate([dst, _N + (ar % (_NPAD - _N))])
  srcw = src.reshape(_NW, k_chunks, _CHUNK)
  dstw = dst.reshape(_NW, k_chunks, _CHUNK)

  sum1, cnt = _make_agg(k_chunks, True)(srcw, dstw, x)

  grid = (_NPAD // _BLK,)
  row_d = pl.BlockSpec((_BLK, _D), lambda i: (i, 0))
  row_l = pl.BlockSpec((_BLK, _LAT), lambda i: (i, 0))
  part = pl.BlockSpec((2, _BLK, _D), lambda i: (0, i, 0))
  cnt_s = pl.BlockSpec((2, _BLK), lambda i: (0, i))
  w_dd = pl.BlockSpec((_D, _D), lambda i: (0, 0))
  w_dl = pl.BlockSpec((_D, _LAT), lambda i: (0, 0))
  v_d = pl.BlockSpec((1, _D), lambda i: (0, 0))
  v_l = pl.BlockSpec((1, _LAT), lambda i: (0, 0))

  tower_w = []
  for (wl1, bl1, wr1, g1, b1, wl2, bl2, wr2) in (
      (Wl1_mu, bl1_mu, Wr1_mu, g1_mu, b1_mu, Wl2_mu, bl2_mu, Wr2_mu),
      (Wl1_lv, bl1_lv, Wr1_lv, g1_lv, b1_lv, Wl2_lv, bl2_lv, Wr2_lv)):
    tower_w += [wl1.T, wr1.T, bl1.reshape(1, _D), g1.reshape(1, _D),
                b1.reshape(1, _D), wl2.T, wr2.T, bl2.reshape(1, _LAT)]
  tower_specs = [w_dd, w_dd, v_d, v_d, v_d, w_dl, w_dl, v_l] * 2

  p, r_mu, r_lv = pl.pallas_call(
      _tc1_body,
      grid=grid,
      in_specs=[part, cnt_s, row_d] + tower_specs,
      out_specs=[row_d, row_l, row_l],
      out_shape=[
          jax.ShapeDtypeStruct((_N, _D), _F32),
          jax.ShapeDtypeStruct((_N, _LAT), _F32),
          jax.ShapeDtypeStruct((_N, _LAT), _F32),
      ],
  )(sum1, cnt, x, *tower_w)

  (sum2,) = _make_agg(k_chunks, False)(srcw, dstw, p)

  mu, lv = pl.pallas_call(
      _tc2_body,
      grid=grid,
      in_specs=[part, cnt_s, row_l, row_l],
      out_specs=[row_l, row_l],
      out_shape=[
          jax.ShapeDtypeStruct((_N, _LAT), _F32),
          jax.ShapeDtypeStruct((_N, _LAT), _F32),
      ],
  )(sum2, cnt, r_mu, r_lv)

  return (mu, lv)
